# baseline (speedup 1.0000x reference)
import functools

import jax
import jax.numpy as jnp
import numpy as np
from jax import lax
from jax.experimental import pallas as pl
from jax.experimental.pallas import tpu as pltpu
from jax.experimental.pallas import tpu_sc as plsc

NC = 2
NS = 16
NW = NC * NS
LANES = 16
CH = 128
NBUF = 3


def kernel(x, W0, W1, W2, W3):
    B, S = x.shape
    V, D = W0.shape
    n_bl = B // CH
    sub = D // 8

    xt = x.T.astype(jnp.int32)

    mesh = plsc.VectorSubcoreMesh(core_axis_name="c", subcore_axis_name="s")

    @functools.partial(
        pl.kernel,
        mesh=mesh,
        out_type=jax.ShapeDtypeStruct((S, 8, n_bl, sub, CH), jnp.float32),
        compiler_params=pltpu.CompilerParams(use_tc_tiling_on_sc=False,
                                             needs_layout_passes=False),
        scratch_types=[
            pltpu.VMEM((S, CH), jnp.int32),
            *([pltpu.VMEM((CH, D), jnp.float32)] * NBUF),
            *([pltpu.VMEM((8, sub, CH + 1), jnp.float32)] * NBUF),
            *([pltpu.SemaphoreType.DMA] * (NBUF * 3)),
        ],
    )
    def sc_avg(x_hbm, w0_hbm, w1_hbm, w2_hbm, w3_hbm, out_hbm,
               idx_v, ac0, ac1, ac2, ob0, ob1, ob2,
               gsem0, gsem1, gsem2, ssem0, ssem1, ssem2,
               tsem0, tsem1, tsem2):
        wid = lax.axis_index("s") * NC + lax.axis_index("c")
        pltpu.sync_copy(x_hbm.at[:, pl.ds(wid * CH, CH)], idx_v)

        tabs = (w0_hbm, w1_hbm, w2_hbm, w3_hbm)
        accs = (ac0, ac1, ac2)
        obufs = (ob0, ob1, ob2)
        gsems = (gsem0, gsem1, gsem2)
        ssems = (ssem0, ssem1, ssem2)
        tsems = (tsem0, tsem1, tsem2)
        zeros = jnp.zeros((LANES,), jnp.float32)
        izeros = jnp.zeros((LANES,), jnp.int32)
        lane = lax.iota(jnp.int32, LANES)
        dlvec = lax.bitwise_and(lane, 7)
        rowbase = lax.shift_right_logical(lane, 3)
        rows = [rowbase + (2 * j) for j in range(D // LANES)]

        def fire_t0(c, s):
            pltpu.async_copy(tabs[0].at[idx_v.at[c]], accs[s], tsems[s])

        def fire_adds(c, s):
            idx = idx_v.at[c]
            for t in range(1, 4):
                pltpu.async_copy(tabs[t].at[idx], accs[s], gsems[s],
                                 add=True)

        def wait_t0(s):
            pltpu.make_async_copy(tabs[0].at[idx_v.at[0]], accs[s],
                                  tsems[s]).wait()

        def wait_adds(s):
            for _ in range(3):
                pltpu.make_async_copy(tabs[0].at[idx_v.at[0]], accs[s],
                                      gsems[s]).wait()

        for s in range(NBUF):
            fire_t0(s, s)
        wait_t0(0)
        fire_adds(0, 0)

        def round_step(c, s):
            @pl.when(c + 1 < S)
            def _():
                wait_t0((c + 1) % NBUF if isinstance(c, int) else 0)

            ac, ob = accs[s], obufs[s]
            wait_adds(s)

            @pl.when(c >= NBUF)
            def _():
                pltpu.make_async_copy(
                    ob.at[:, :, pl.ds(0, CH)], out_hbm.at[0, :, 0],
                    ssems[s]).wait()

            def row_body(i, carry2):
                cols = izeros + i
                for j in range(D // LANES):
                    sl = pl.ds(j * LANES, LANES)
                    plsc.store_scatter(ob, [rows[j], dlvec, cols],
                                       ac[i, sl] * 0.25)
                return carry2

            lax.fori_loop(0, CH, row_body, 0, unroll=4)

            pltpu.make_async_copy(
                ob.at[:, :, pl.ds(0, CH)], out_hbm.at[c, :, wid],
                ssems[s]).start()

            @pl.when(c + NBUF < S)
            def _():
                fire_t0(c + NBUF, s)

        def trip_body(p, carry):
            for s in range(NBUF):
                c = p * NBUF + s

                @pl.when(c + 1 < S)
                def _():
                    wait_t0((s + 1) % NBUF)
                    fire_adds(c + 1, (s + 1) % NBUF)

                round_step_inner(c, s)
            return carry

        def round_step_inner(c, s):
            ac, ob = accs[s], obufs[s]
            wait_adds(s)

            @pl.when(c >= NBUF)
            def _():
                pltpu.make_async_copy(
                    ob.at[:, :, pl.ds(0, CH)], out_hbm.at[0, :, 0],
                    ssems[s]).wait()

            def row_body(i, carry2):
                cols = izeros + i
                for j in range(D // LANES):
                    sl = pl.ds(j * LANES, LANES)
                    plsc.store_scatter(ob, [rows[j], dlvec, cols],
                                       ac[i, sl] * 0.25)
                return carry2

            lax.fori_loop(0, CH, row_body, 0, unroll=4)

            pltpu.make_async_copy(
                ob.at[:, :, pl.ds(0, CH)], out_hbm.at[c, :, wid],
                ssems[s]).start()

            @pl.when(c + NBUF < S)
            def _():
                fire_t0(c + NBUF, s)

        lax.fori_loop(0, S // NBUF, trip_body, 0)
        for r in range(S - S % NBUF, S):
            s = r % NBUF

            @pl.when(r + 1 < S)
            def _():
                wait_t0((s + 1) % NBUF)
                fire_adds(r + 1, (s + 1) % NBUF)

            round_step_inner(r, s)
        for s in range(NBUF):
            pltpu.make_async_copy(
                obufs[s].at[:, :, pl.ds(0, CH)], out_hbm.at[0, :, 0],
                ssems[s]).wait()

    out5 = sc_avg(xt, W0, W1, W2, W3)
    out = out5.transpose(2, 4, 0, 1, 3).reshape(B, S, D)
    return out


# speedup vs baseline: 1.0071x; 1.0071x over previous
import functools

import jax
import jax.numpy as jnp
import numpy as np
from jax import lax
from jax.experimental import pallas as pl
from jax.experimental.pallas import tpu as pltpu
from jax.experimental.pallas import tpu_sc as plsc

NC = 2
NS = 16
NW = NC * NS
LANES = 16
CH = 128
NBUF = 2


def kernel(x, W0, W1, W2, W3):
    B, S = x.shape
    V, D = W0.shape
    n_bl = B // CH
    sub = D // 8

    xt = x.T.astype(jnp.int32)

    mesh = plsc.VectorSubcoreMesh(core_axis_name="c", subcore_axis_name="s")

    @functools.partial(
        pl.kernel,
        mesh=mesh,
        out_type=jax.ShapeDtypeStruct((S, 8, n_bl, sub, CH), jnp.float32),
        compiler_params=pltpu.CompilerParams(use_tc_tiling_on_sc=False,
                                             needs_layout_passes=False),
        scratch_types=[
            pltpu.VMEM((S, CH), jnp.int32),
            *([pltpu.VMEM((CH, D), jnp.float32)] * NBUF),
            *([pltpu.VMEM((8, sub, CH + 1), jnp.float32)] * NBUF),
            *([pltpu.SemaphoreType.DMA] * (NBUF * 2)),
        ],
    )
    def sc_avg(x_hbm, w0_hbm, w1_hbm, w2_hbm, w3_hbm, out_hbm,
               idx_v, ac0, ac1, ob0, ob1, gsem0, gsem1, ssem0, ssem1):
        wid = lax.axis_index("s") * NC + lax.axis_index("c")
        pltpu.sync_copy(x_hbm.at[:, pl.ds(wid * CH, CH)], idx_v)

        tabs = (w0_hbm, w1_hbm, w2_hbm, w3_hbm)
        accs = (ac0, ac1)
        obufs = (ob0, ob1)
        gsems = (gsem0, gsem1)
        ssems = (ssem0, ssem1)
        zeros = jnp.zeros((LANES,), jnp.float32)
        izeros = jnp.zeros((LANES,), jnp.int32)
        lane = lax.iota(jnp.int32, LANES)
        dlvec = lax.bitwise_and(lane, 7)
        rowbase = lax.shift_right_logical(lane, 3)
        rows = [rowbase + (2 * j) for j in range(D // LANES)]

        def zero_acc(ac):
            def zbody(i, carry):
                for j in range(D // LANES):
                    ac[i, pl.ds(j * LANES, LANES)] = zeros
                return carry
            lax.fori_loop(0, CH, zbody, 0, unroll=8)

        def fire(c, s):
            idx = idx_v.at[c]
            for t in range(4):
                pltpu.async_copy(tabs[t].at[idx], accs[s], gsems[s],
                                 add=True)

        for s in range(NBUF):
            zero_acc(accs[s])
            fire(s, s)

        def pair_body(p, carry):
            for s in range(NBUF):
                c = p * NBUF + s
                ac, ob = accs[s], obufs[s]
                idx0 = idx_v.at[0]
                for _ in range(4):
                    pltpu.make_async_copy(tabs[0].at[idx0], ac,
                                          gsems[s]).wait()

                @pl.when(c >= NBUF)
                def _():
                    pltpu.make_async_copy(
                        ob.at[:, :, pl.ds(0, CH)], out_hbm.at[0, :, 0],
                        ssems[s]).wait()

                def row_body(i, carry2):
                    cols = izeros + i
                    for j in range(D // LANES):
                        sl = pl.ds(j * LANES, LANES)
                        v = ac[i, sl]
                        ac[i, sl] = zeros
                        plsc.store_scatter(ob, [rows[j], dlvec, cols],
                                           v * 0.25)
                    return carry2

                lax.fori_loop(0, CH, row_body, 0, unroll=4)

                pltpu.make_async_copy(
                    ob.at[:, :, pl.ds(0, CH)], out_hbm.at[c, :, wid],
                    ssems[s]).start()

                @pl.when(c + NBUF < S)
                def _():
                    fire(c + NBUF, s)
            return carry

        lax.fori_loop(0, S // NBUF, pair_body, 0)
        for s in range(NBUF):
            pltpu.make_async_copy(
                obufs[s].at[:, :, pl.ds(0, CH)], out_hbm.at[0, :, 0],
                ssems[s]).wait()

    out5 = sc_avg(xt, W0, W1, W2, W3)
    out = out5.transpose(2, 4, 0, 1, 3).reshape(B, S, D)
    return out
